# baseline (speedup 1.0000x reference)
import functools

import jax
import jax.numpy as jnp
from jax import lax
from jax.experimental import pallas as pl
from jax.experimental.pallas import tpu as pltpu
from jax.experimental.pallas import tpu_sc as plsc

N_NODES = 100000
WALK_LENGTH = 16
EMBED_DIM = 16

NSC = 2
NTILES = 16
NW = NSC * NTILES
L = 16
NPAD = 102400
SLICE = NPAD // NTILES
WSLICE = NPAD // NW
CHUNK = 1280
VPC = CHUNK // L
NCH = 80
NPAIR = NCH // 2
EPT = CHUNK * NCH
E_PAD = EPT * NW
E_ALLOC = E_PAD + 2 * CHUNK
STEPS = WALK_LENGTH - 1
NVS = SLICE // L
NVW = WSLICE // L
FWORDS = 8 * NW


def _rsqrt_nr(d):
    i = lax.bitcast_convert_type(d, jnp.int32)
    i = jnp.int32(0x5F3759DF) - (i >> 1)
    y = lax.bitcast_convert_type(i, jnp.float32)
    for _ in range(3):
        y = y * (1.5 - 0.5 * d * y * y)
    return y


def _walk_body(row_hbm, col_hbm, flags_hbm, probs_hbm, q_hbm, part_hbm,
               q_local, rowb_a, colb_a, valb_a, rowb_b, colb_b, valb_b,
               accs, bufb, dinvs, flagv, flagall, acc_sh,
               sem_ld_a, sem_ld_b, sem_sc_a, sem_sc_b):
    cid = lax.axis_index("c")
    tid = lax.axis_index("s")
    wid = cid * NTILES + tid
    sbase = tid * SLICE
    wbase = wid * WSLICE
    ebase = wid * EPT
    zeros16 = jnp.zeros((L,), jnp.float32)
    inv_n = jnp.float32(1.0 / N_NODES)

    def _gbarrier(k):
        flagv[...] = jnp.zeros((L,), jnp.int32) + k
        pltpu.sync_copy(flagv.at[pl.ds(0, 8)], flags_hbm.at[pl.ds(wid * 8, 8)])

        def _poll(_):
            pltpu.sync_copy(flags_hbm, flagall)
            m = flagall[pl.ds(0, L)]
            for r in range(1, FWORDS // L):
                m = jnp.minimum(m, flagall[pl.ds(r * L, L)])
            return jnp.min(m)
        lax.while_loop(lambda m: m < k, _poll, jnp.int32(0))

    def _zero_slice(i, _):
        accs[pl.ds(i * L, L)] = zeros16
        return 0

    def _zero_acc_and_publish():
        lax.fori_loop(0, NVS, _zero_slice, 0)
        pltpu.sync_copy(accs, acc_sh.at[pl.ds(sbase, SLICE)])

    def _eb(c):
        return pl.multiple_of(ebase + c * CHUNK, 8)

    def _prime():
        def _fill(i, _):
            colb_b[pl.ds(i * L, L)] = jnp.full((L,), NPAD - 1, jnp.int32)
            return 0
        lax.fori_loop(0, VPC, _fill, 0)
        pltpu.async_copy(valb_b, acc_sh.at[colb_b], sem_sc_b, add=True)

    def _gather(rb, vb):
        @plsc.parallel_loop(0, VPC, unroll=4)
        def _g(i):
            o = i * L
            vb[pl.ds(o, L)] = plsc.load_gather(q_local, [rb[pl.ds(o, L)]])

    _zero_acc_and_publish()
    plsc.subcore_barrier()

    _prime()
    pltpu.async_copy(col_hbm.at[pl.ds(_eb(0), CHUNK)], colb_a, sem_ld_a)

    def _ones_body(i, _):
        valb_a[pl.ds(i * L, L)] = zeros16 + 1.0
        valb_b[pl.ds(i * L, L)] = zeros16 + 1.0
        return 0
    lax.fori_loop(0, VPC, _ones_body, 0)

    def _deg_pair(p, _):
        c0 = 2 * p
        pltpu.make_async_copy(
            col_hbm.at[pl.ds(_eb(c0), CHUNK)], colb_a, sem_ld_a).wait()
        pltpu.make_async_copy(valb_b, acc_sh.at[colb_b], sem_sc_b).wait()
        pltpu.async_copy(col_hbm.at[pl.ds(_eb(c0 + 1), CHUNK)], colb_b, sem_ld_b)
        pltpu.async_copy(valb_a, acc_sh.at[colb_a], sem_sc_a, add=True)
        pltpu.make_async_copy(
            col_hbm.at[pl.ds(_eb(c0 + 1), CHUNK)], colb_b, sem_ld_b).wait()
        pltpu.make_async_copy(valb_a, acc_sh.at[colb_a], sem_sc_a).wait()
        pltpu.async_copy(col_hbm.at[pl.ds(_eb(c0 + 2), CHUNK)], colb_a, sem_ld_a)
        pltpu.async_copy(valb_b, acc_sh.at[colb_b], sem_sc_b, add=True)
        return 0
    lax.fori_loop(0, NPAIR, _deg_pair, 0)
    pltpu.make_async_copy(
        col_hbm.at[pl.ds(_eb(NCH), CHUNK)], colb_a, sem_ld_a).wait()
    pltpu.make_async_copy(valb_b, acc_sh.at[colb_b], sem_sc_b).wait()
    plsc.subcore_barrier()
    pltpu.sync_copy(acc_sh.at[pl.ds(sbase, SLICE)],
                    part_hbm.at[pl.ds(cid * NPAD + sbase, SLICE)])
    _gbarrier(1)

    pltpu.sync_copy(part_hbm.at[pl.ds(wbase, WSLICE)], accs.at[pl.ds(0, WSLICE)])
    pltpu.sync_copy(part_hbm.at[pl.ds(NPAD + wbase, WSLICE)], bufb)

    def _init_body(i, _):
        deg = accs[pl.ds(i * L, L)] + bufb[pl.ds(i * L, L)] + 1.0
        dinvs[pl.ds(i * L, L)] = _rsqrt_nr(deg)
        accs[pl.ds(i * L, L)] = zeros16 + inv_n
        return 0
    lax.fori_loop(0, NVW, _init_body, 0)
    pltpu.sync_copy(accs.at[pl.ds(0, WSLICE)], probs_hbm.at[pl.ds(wbase, WSLICE)])

    def _q0_body(i, _):
        accs[pl.ds(i * L, L)] = dinvs[pl.ds(i * L, L)] * inv_n
        return 0
    lax.fori_loop(0, NVW, _q0_body, 0)
    pltpu.sync_copy(accs.at[pl.ds(0, WSLICE)], q_hbm.at[pl.ds(wbase, WSLICE)])
    _zero_acc_and_publish()
    _gbarrier(2)
    pltpu.sync_copy(q_hbm, q_local)

    def _step_body(t, _):
        _prime()
        pltpu.async_copy(row_hbm.at[pl.ds(_eb(0), CHUNK)], rowb_a, sem_ld_a)
        pltpu.async_copy(col_hbm.at[pl.ds(_eb(0), CHUNK)], colb_a, sem_ld_a)

        def _pair(p, _p):
            c0 = 2 * p
            pltpu.make_async_copy(
                row_hbm.at[pl.ds(_eb(c0), CHUNK)], rowb_a, sem_ld_a).wait()
            pltpu.make_async_copy(
                col_hbm.at[pl.ds(_eb(c0), CHUNK)], colb_a, sem_ld_a).wait()
            pltpu.make_async_copy(valb_b, acc_sh.at[colb_b], sem_sc_b).wait()
            pltpu.async_copy(row_hbm.at[pl.ds(_eb(c0 + 1), CHUNK)], rowb_b, sem_ld_b)
            pltpu.async_copy(col_hbm.at[pl.ds(_eb(c0 + 1), CHUNK)], colb_b, sem_ld_b)
            _gather(rowb_a, valb_a)
            pltpu.async_copy(valb_a, acc_sh.at[colb_a], sem_sc_a, add=True)
            pltpu.make_async_copy(
                row_hbm.at[pl.ds(_eb(c0 + 1), CHUNK)], rowb_b, sem_ld_b).wait()
            pltpu.make_async_copy(
                col_hbm.at[pl.ds(_eb(c0 + 1), CHUNK)], colb_b, sem_ld_b).wait()
            _gather(rowb_b, valb_b)
            pltpu.make_async_copy(valb_a, acc_sh.at[colb_a], sem_sc_a).wait()
            pltpu.async_copy(row_hbm.at[pl.ds(_eb(c0 + 2), CHUNK)], rowb_a, sem_ld_a)
            pltpu.async_copy(col_hbm.at[pl.ds(_eb(c0 + 2), CHUNK)], colb_a, sem_ld_a)
            pltpu.async_copy(valb_b, acc_sh.at[colb_b], sem_sc_b, add=True)
            return 0
        lax.fori_loop(0, NPAIR, _pair, 0)
        pltpu.make_async_copy(
            row_hbm.at[pl.ds(_eb(NCH), CHUNK)], rowb_a, sem_ld_a).wait()
        pltpu.make_async_copy(
            col_hbm.at[pl.ds(_eb(NCH), CHUNK)], colb_a, sem_ld_a).wait()
        pltpu.make_async_copy(valb_b, acc_sh.at[colb_b], sem_sc_b).wait()
        plsc.subcore_barrier()
        pltpu.sync_copy(acc_sh.at[pl.ds(sbase, SLICE)],
                        part_hbm.at[pl.ds(cid * NPAD + sbase, SLICE)])
        _gbarrier(3 + 2 * t)

        pltpu.sync_copy(part_hbm.at[pl.ds(wbase, WSLICE)],
                        accs.at[pl.ds(0, WSLICE)])
        pltpu.sync_copy(part_hbm.at[pl.ds(NPAD + wbase, WSLICE)], bufb)

        def _upd_body(i, _u):
            s = accs[pl.ds(i * L, L)] + bufb[pl.ds(i * L, L)]
            qv = q_local[pl.ds(wbase + i * L, L)]
            dv = dinvs[pl.ds(i * L, L)]
            pv = qv / dv
            accs[pl.ds(i * L, L)] = 0.9 * (dv * (s + qv)) + 0.1 * pv
            return 0
        lax.fori_loop(0, NVW, _upd_body, 0)
        off = pl.multiple_of((t + 1) * NPAD + wbase, 8)
        pltpu.sync_copy(accs.at[pl.ds(0, WSLICE)], probs_hbm.at[pl.ds(off, WSLICE)])

        def _qn_body(i, _u):
            accs[pl.ds(i * L, L)] = accs[pl.ds(i * L, L)] * dinvs[pl.ds(i * L, L)]
            return 0
        lax.fori_loop(0, NVW, _qn_body, 0)
        pltpu.sync_copy(accs.at[pl.ds(0, WSLICE)], q_hbm.at[pl.ds(wbase, WSLICE)])
        _zero_acc_and_publish()
        _gbarrier(4 + 2 * t)
        pltpu.sync_copy(q_hbm, q_local)
        return 0
    lax.fori_loop(0, STEPS, _step_body, 0)


_walk = functools.partial(
    pl.kernel,
    out_type=[
        jax.ShapeDtypeStruct((WALK_LENGTH * NPAD,), jnp.float32),
        jax.ShapeDtypeStruct((NPAD,), jnp.float32),
        jax.ShapeDtypeStruct((NSC * NPAD,), jnp.float32),
    ],
    mesh=plsc.VectorSubcoreMesh(core_axis_name="c", subcore_axis_name="s"),
    compiler_params=pltpu.CompilerParams(needs_layout_passes=False),
    scratch_types=[
        pltpu.VMEM((NPAD,), jnp.float32),
        pltpu.VMEM((CHUNK,), jnp.int32),
        pltpu.VMEM((CHUNK,), jnp.int32),
        pltpu.VMEM((CHUNK,), jnp.float32),
        pltpu.VMEM((CHUNK,), jnp.int32),
        pltpu.VMEM((CHUNK,), jnp.int32),
        pltpu.VMEM((CHUNK,), jnp.float32),
        pltpu.VMEM((SLICE,), jnp.float32),
        pltpu.VMEM((WSLICE,), jnp.float32),
        pltpu.VMEM((WSLICE,), jnp.float32),
        pltpu.VMEM((L,), jnp.int32),
        pltpu.VMEM((FWORDS,), jnp.int32),
        pltpu.VMEM_SHARED((NPAD,), jnp.float32),
        pltpu.SemaphoreType.DMA,
        pltpu.SemaphoreType.DMA,
        pltpu.SemaphoreType.DMA,
        pltpu.SemaphoreType.DMA,
    ],
)(_walk_body)


BN = 2048


def _linear_body(p_ref, w_ref, b_ref, o_ref):
    o_ref[...] = lax.dot_general(
        p_ref[...], w_ref[...], (((0,), (1,)), ((), ())),
        preferred_element_type=jnp.float32) + b_ref[...]


def _linear(probs2d, W, b2d):
    return pl.pallas_call(
        _linear_body,
        grid=(NPAD // BN,),
        in_specs=[
            pl.BlockSpec((WALK_LENGTH, BN), lambda i: (0, i)),
            pl.BlockSpec((EMBED_DIM, WALK_LENGTH), lambda i: (0, 0)),
            pl.BlockSpec((1, EMBED_DIM), lambda i: (0, 0)),
        ],
        out_specs=pl.BlockSpec((BN, EMBED_DIM), lambda i: (i, 0)),
        out_shape=jax.ShapeDtypeStruct((NPAD, EMBED_DIM), jnp.float32),
    )(probs2d, W, b2d)


def _zeroflags_body(o_ref):
    o_ref[...] = jnp.zeros((1, FWORDS), jnp.int32)


def _zeroflags():
    out = pl.pallas_call(
        _zeroflags_body,
        out_shape=jax.ShapeDtypeStruct((1, FWORDS), jnp.int32),
    )()
    return out.reshape(FWORDS)


def kernel(edge_index, num_nodes, W, b):
    ei = edge_index.astype(jnp.int32)
    row, col = ei[0], ei[1]
    pad = E_ALLOC - row.shape[0]
    row_p = jnp.concatenate([row, jnp.zeros((pad,), jnp.int32)])
    col_p = jnp.concatenate([col, jnp.full((pad,), N_NODES, jnp.int32)])
    flags0 = _zeroflags()
    probs_flat, _, _ = _walk(row_p, col_p, flags0)
    probs2d = probs_flat.reshape(WALK_LENGTH, NPAD)
    out = _linear(probs2d, W.astype(jnp.float32),
                  b.astype(jnp.float32).reshape(1, EMBED_DIM))
    return out[:N_NODES]


# speedup vs baseline: 1.0001x; 1.0001x over previous
import functools

import jax
import jax.numpy as jnp
from jax import lax
from jax.experimental import pallas as pl
from jax.experimental.pallas import tpu as pltpu
from jax.experimental.pallas import tpu_sc as plsc

N_NODES = 100000
WALK_LENGTH = 16
EMBED_DIM = 16

NSC = 2
NTILES = 16
NW = NSC * NTILES
L = 16
NPAD = 102400
SLICE = NPAD // NTILES
WSLICE = NPAD // NW
CHUNK = 1280
VPC = CHUNK // L
NCH = 80
NPAIR = NCH // 2
EPT = CHUNK * NCH
E_PAD = EPT * NW
E_ALLOC = E_PAD + 2 * CHUNK
STEPS = WALK_LENGTH - 1
NVS = SLICE // L
NVW = WSLICE // L
FWORDS = L * NW


def _rsqrt_nr(d):
    i = lax.bitcast_convert_type(d, jnp.int32)
    i = jnp.int32(0x5F3759DF) - (i >> 1)
    y = lax.bitcast_convert_type(i, jnp.float32)
    for _ in range(3):
        y = y * (1.5 - 0.5 * d * y * y)
    return y


def _walk_body(row_hbm, col_hbm, flags_hbm, probs_hbm, q_hbm, part_hbm,
               q_local, rowb_a, colb_a, valb_a, rowb_b, colb_b, valb_b,
               accs, bufb, dinvs, flagv, flagall, acc_sh,
               sem_ld_a, sem_ld_b, sem_sc_a, sem_sc_b):
    cid = lax.axis_index("c")
    tid = lax.axis_index("s")
    wid = cid * NTILES + tid
    sbase = tid * SLICE
    wbase = wid * WSLICE
    ebase = wid * EPT
    zeros16 = jnp.zeros((L,), jnp.float32)
    inv_n = jnp.float32(1.0 / N_NODES)

    def _gbarrier(k):
        flagv[...] = jnp.zeros((L,), jnp.int32) + k
        pltpu.sync_copy(flagv, flags_hbm.at[pl.ds(wid * L, L)])

        def _poll(_):
            pltpu.sync_copy(flags_hbm, flagall)
            m = flagall[pl.ds(0, L)]
            for r in range(1, FWORDS // L):
                m = jnp.minimum(m, flagall[pl.ds(r * L, L)])
            return jnp.min(m)
        lax.while_loop(lambda m: m < k, _poll, jnp.int32(0))

    def _zero_slice(i, _):
        accs[pl.ds(i * L, L)] = zeros16
        return 0

    def _zero_acc_and_publish():
        lax.fori_loop(0, NVS, _zero_slice, 0)
        pltpu.sync_copy(accs, acc_sh.at[pl.ds(sbase, SLICE)])

    def _eb(c):
        return pl.multiple_of(ebase + c * CHUNK, 8)

    def _prime():
        def _fill(i, _):
            colb_b[pl.ds(i * L, L)] = jnp.full((L,), NPAD - 1, jnp.int32)
            return 0
        lax.fori_loop(0, VPC, _fill, 0)
        pltpu.async_copy(valb_b, acc_sh.at[colb_b], sem_sc_b, add=True)

    def _gather(rb, vb):
        @plsc.parallel_loop(0, VPC, unroll=4)
        def _g(i):
            o = i * L
            vb[pl.ds(o, L)] = plsc.load_gather(q_local, [rb[pl.ds(o, L)]])

    _zero_acc_and_publish()
    plsc.subcore_barrier()

    _prime()
    pltpu.async_copy(col_hbm.at[pl.ds(_eb(0), CHUNK)], colb_a, sem_ld_a)

    def _ones_body(i, _):
        valb_a[pl.ds(i * L, L)] = zeros16 + 1.0
        valb_b[pl.ds(i * L, L)] = zeros16 + 1.0
        return 0
    lax.fori_loop(0, VPC, _ones_body, 0)

    def _deg_pair(p, _):
        c0 = 2 * p
        pltpu.make_async_copy(
            col_hbm.at[pl.ds(_eb(c0), CHUNK)], colb_a, sem_ld_a).wait()
        pltpu.make_async_copy(valb_b, acc_sh.at[colb_b], sem_sc_b).wait()
        pltpu.async_copy(col_hbm.at[pl.ds(_eb(c0 + 1), CHUNK)], colb_b, sem_ld_b)
        pltpu.async_copy(valb_a, acc_sh.at[colb_a], sem_sc_a, add=True)
        pltpu.make_async_copy(
            col_hbm.at[pl.ds(_eb(c0 + 1), CHUNK)], colb_b, sem_ld_b).wait()
        pltpu.make_async_copy(valb_a, acc_sh.at[colb_a], sem_sc_a).wait()
        pltpu.async_copy(col_hbm.at[pl.ds(_eb(c0 + 2), CHUNK)], colb_a, sem_ld_a)
        pltpu.async_copy(valb_b, acc_sh.at[colb_b], sem_sc_b, add=True)
        return 0
    lax.fori_loop(0, NPAIR, _deg_pair, 0)
    pltpu.make_async_copy(
        col_hbm.at[pl.ds(_eb(NCH), CHUNK)], colb_a, sem_ld_a).wait()
    pltpu.make_async_copy(valb_b, acc_sh.at[colb_b], sem_sc_b).wait()
    plsc.subcore_barrier()
    pltpu.sync_copy(acc_sh.at[pl.ds(sbase, SLICE)],
                    part_hbm.at[pl.ds(cid * NPAD + sbase, SLICE)])
    _gbarrier(1)

    pltpu.sync_copy(part_hbm.at[pl.ds(wbase, WSLICE)], accs.at[pl.ds(0, WSLICE)])
    pltpu.sync_copy(part_hbm.at[pl.ds(NPAD + wbase, WSLICE)], bufb)

    def _init_body(i, _):
        deg = accs[pl.ds(i * L, L)] + bufb[pl.ds(i * L, L)] + 1.0
        dinvs[pl.ds(i * L, L)] = _rsqrt_nr(deg)
        accs[pl.ds(i * L, L)] = zeros16 + inv_n
        return 0
    lax.fori_loop(0, NVW, _init_body, 0)
    pltpu.sync_copy(accs.at[pl.ds(0, WSLICE)], probs_hbm.at[pl.ds(wbase, WSLICE)])

    def _q0_body(i, _):
        accs[pl.ds(i * L, L)] = dinvs[pl.ds(i * L, L)] * inv_n
        return 0
    lax.fori_loop(0, NVW, _q0_body, 0)
    pltpu.sync_copy(accs.at[pl.ds(0, WSLICE)], q_hbm.at[pl.ds(wbase, WSLICE)])
    _zero_acc_and_publish()
    _gbarrier(2)
    pltpu.sync_copy(q_hbm, q_local)

    def _step_body(t, _):
        _prime()
        pltpu.async_copy(row_hbm.at[pl.ds(_eb(0), CHUNK)], rowb_a, sem_ld_a)
        pltpu.async_copy(col_hbm.at[pl.ds(_eb(0), CHUNK)], colb_a, sem_ld_a)

        def _pair(p, _p):
            c0 = 2 * p
            pltpu.make_async_copy(
                row_hbm.at[pl.ds(_eb(c0), CHUNK)], rowb_a, sem_ld_a).wait()
            pltpu.make_async_copy(
                col_hbm.at[pl.ds(_eb(c0), CHUNK)], colb_a, sem_ld_a).wait()
            pltpu.make_async_copy(valb_b, acc_sh.at[colb_b], sem_sc_b).wait()
            pltpu.async_copy(row_hbm.at[pl.ds(_eb(c0 + 1), CHUNK)], rowb_b, sem_ld_b)
            pltpu.async_copy(col_hbm.at[pl.ds(_eb(c0 + 1), CHUNK)], colb_b, sem_ld_b)
            _gather(rowb_a, valb_a)
            pltpu.async_copy(valb_a, acc_sh.at[colb_a], sem_sc_a, add=True)
            pltpu.make_async_copy(
                row_hbm.at[pl.ds(_eb(c0 + 1), CHUNK)], rowb_b, sem_ld_b).wait()
            pltpu.make_async_copy(
                col_hbm.at[pl.ds(_eb(c0 + 1), CHUNK)], colb_b, sem_ld_b).wait()
            _gather(rowb_b, valb_b)
            pltpu.make_async_copy(valb_a, acc_sh.at[colb_a], sem_sc_a).wait()
            pltpu.async_copy(row_hbm.at[pl.ds(_eb(c0 + 2), CHUNK)], rowb_a, sem_ld_a)
            pltpu.async_copy(col_hbm.at[pl.ds(_eb(c0 + 2), CHUNK)], colb_a, sem_ld_a)
            pltpu.async_copy(valb_b, acc_sh.at[colb_b], sem_sc_b, add=True)
            return 0
        lax.fori_loop(0, NPAIR, _pair, 0)
        pltpu.make_async_copy(
            row_hbm.at[pl.ds(_eb(NCH), CHUNK)], rowb_a, sem_ld_a).wait()
        pltpu.make_async_copy(
            col_hbm.at[pl.ds(_eb(NCH), CHUNK)], colb_a, sem_ld_a).wait()
        pltpu.make_async_copy(valb_b, acc_sh.at[colb_b], sem_sc_b).wait()
        plsc.subcore_barrier()
        pltpu.sync_copy(acc_sh.at[pl.ds(sbase, SLICE)],
                        part_hbm.at[pl.ds(cid * NPAD + sbase, SLICE)])
        _gbarrier(3 + 2 * t)

        pltpu.sync_copy(part_hbm.at[pl.ds(wbase, WSLICE)],
                        accs.at[pl.ds(0, WSLICE)])
        pltpu.sync_copy(part_hbm.at[pl.ds(NPAD + wbase, WSLICE)], bufb)

        def _upd_body(i, _u):
            s = accs[pl.ds(i * L, L)] + bufb[pl.ds(i * L, L)]
            qv = q_local[pl.ds(wbase + i * L, L)]
            dv = dinvs[pl.ds(i * L, L)]
            pv = qv / dv
            accs[pl.ds(i * L, L)] = 0.9 * (dv * (s + qv)) + 0.1 * pv
            return 0
        lax.fori_loop(0, NVW, _upd_body, 0)
        off = pl.multiple_of((t + 1) * NPAD + wbase, 8)
        pltpu.sync_copy(accs.at[pl.ds(0, WSLICE)], probs_hbm.at[pl.ds(off, WSLICE)])

        def _qn_body(i, _u):
            accs[pl.ds(i * L, L)] = accs[pl.ds(i * L, L)] * dinvs[pl.ds(i * L, L)]
            return 0
        lax.fori_loop(0, NVW, _qn_body, 0)
        pltpu.sync_copy(accs.at[pl.ds(0, WSLICE)], q_hbm.at[pl.ds(wbase, WSLICE)])
        _zero_acc_and_publish()
        _gbarrier(4 + 2 * t)
        pltpu.sync_copy(q_hbm, q_local)
        return 0
    lax.fori_loop(0, STEPS, _step_body, 0)


_walk = functools.partial(
    pl.kernel,
    out_type=[
        jax.ShapeDtypeStruct((WALK_LENGTH * NPAD,), jnp.float32),
        jax.ShapeDtypeStruct((NPAD,), jnp.float32),
        jax.ShapeDtypeStruct((NSC * NPAD,), jnp.float32),
    ],
    mesh=plsc.VectorSubcoreMesh(core_axis_name="c", subcore_axis_name="s"),
    compiler_params=pltpu.CompilerParams(needs_layout_passes=False),
    scratch_types=[
        pltpu.VMEM((NPAD,), jnp.float32),
        pltpu.VMEM((CHUNK,), jnp.int32),
        pltpu.VMEM((CHUNK,), jnp.int32),
        pltpu.VMEM((CHUNK,), jnp.float32),
        pltpu.VMEM((CHUNK,), jnp.int32),
        pltpu.VMEM((CHUNK,), jnp.int32),
        pltpu.VMEM((CHUNK,), jnp.float32),
        pltpu.VMEM((SLICE,), jnp.float32),
        pltpu.VMEM((WSLICE,), jnp.float32),
        pltpu.VMEM((WSLICE,), jnp.float32),
        pltpu.VMEM((L,), jnp.int32),
        pltpu.VMEM((FWORDS,), jnp.int32),
        pltpu.VMEM_SHARED((NPAD,), jnp.float32),
        pltpu.SemaphoreType.DMA,
        pltpu.SemaphoreType.DMA,
        pltpu.SemaphoreType.DMA,
        pltpu.SemaphoreType.DMA,
    ],
)(_walk_body)


BN = 2048


def _linear_body(p_ref, w_ref, b_ref, o_ref):
    o_ref[...] = lax.dot_general(
        p_ref[...], w_ref[...], (((0,), (1,)), ((), ())),
        preferred_element_type=jnp.float32) + b_ref[...]


def _linear(probs2d, W, b2d):
    return pl.pallas_call(
        _linear_body,
        grid=(NPAD // BN,),
        in_specs=[
            pl.BlockSpec((WALK_LENGTH, BN), lambda i: (0, i)),
            pl.BlockSpec((EMBED_DIM, WALK_LENGTH), lambda i: (0, 0)),
            pl.BlockSpec((1, EMBED_DIM), lambda i: (0, 0)),
        ],
        out_specs=pl.BlockSpec((BN, EMBED_DIM), lambda i: (i, 0)),
        out_shape=jax.ShapeDtypeStruct((NPAD, EMBED_DIM), jnp.float32),
    )(probs2d, W, b2d)


def _zeroflags_body(o_ref):
    o_ref[...] = jnp.zeros((1, FWORDS), jnp.int32)


def _zeroflags():
    out = pl.pallas_call(
        _zeroflags_body,
        out_shape=jax.ShapeDtypeStruct((1, FWORDS), jnp.int32),
    )()
    return out.reshape(FWORDS)


def kernel(edge_index, num_nodes, W, b):
    ei = edge_index.astype(jnp.int32)
    row, col = ei[0], ei[1]
    pad = E_ALLOC - row.shape[0]
    row_p = jnp.concatenate([row, jnp.zeros((pad,), jnp.int32)])
    col_p = jnp.concatenate([col, jnp.full((pad,), N_NODES, jnp.int32)])
    flags0 = _zeroflags()
    probs_flat, _, _ = _walk(row_p, col_p, flags0)
    probs2d = probs_flat.reshape(WALK_LENGTH, NPAD)
    out = _linear(probs2d, W.astype(jnp.float32),
                  b.astype(jnp.float32).reshape(1, EMBED_DIM))
    return out[:N_NODES]


# speedup vs baseline: 1.3201x; 1.3199x over previous
import functools

import jax
import jax.numpy as jnp
from jax import lax
from jax.experimental import pallas as pl
from jax.experimental.pallas import tpu as pltpu
from jax.experimental.pallas import tpu_sc as plsc

N_NODES = 100000
WALK_LENGTH = 16
EMBED_DIM = 16

NSC = 2
NTILES = 16
NW = NSC * NTILES
L = 16
NPAD = 102400
SLICE = NPAD // NTILES
WSLICE = NPAD // NW
CHUNK = 1024
VPC = CHUNK // L
NCH = 98
NPAIR = NCH // 2
EPT = CHUNK * NCH
E_PAD = EPT * NW
E_ALLOC = E_PAD + 2 * CHUNK
STEPS = WALK_LENGTH - 1
NVS = SLICE // L
NVW = WSLICE // L
FROWS = NW * L


def _rsqrt_nr(d):
    i = lax.bitcast_convert_type(d, jnp.int32)
    i = jnp.int32(0x5F3759DF) - (i >> 1)
    y = lax.bitcast_convert_type(i, jnp.float32)
    for _ in range(3):
        y = y * (1.5 - 0.5 * d * y * y)
    return y


def _walk_body(row_hbm, col_hbm, flags_hbm, probs_hbm, q_hbm, part_hbm,
               q_local, rowb_a, colb_a, valb_a, rowb_b, colb_b, valb_b,
               accs, bufb, dinvs, flagv, flagall, acc_sh,
               sem_ld_a, sem_ld_b, sem_sc_a, sem_sc_b):
    cid = lax.axis_index("c")
    tid = lax.axis_index("s")
    wid = cid * NTILES + tid
    sbase = tid * SLICE
    wbase = wid * WSLICE
    ebase = wid * EPT
    zeros16 = jnp.zeros((L,), jnp.float32)
    inv_n = jnp.float32(1.0 / N_NODES)

    def _gbarrier(k):
        flagv[...] = jnp.zeros((L,), jnp.int32) + k
        pltpu.sync_copy(flagv, flags_hbm.at[pl.ds(wid * L, L)])

        def _poll(_):
            pltpu.sync_copy(flags_hbm, flagall)
            m = flagall[pl.ds(0, L)]
            for r in range(1, NW):
                m = jnp.minimum(m, flagall[pl.ds(r * L, L)])
            return jnp.min(m)
        lax.while_loop(lambda m: m < k, _poll, jnp.int32(0))

    def _zero_slice(i, _):
        accs[pl.ds(i * L, L)] = zeros16
        return 0

    def _zero_acc_and_publish():
        lax.fori_loop(0, NVS, _zero_slice, 0)
        pltpu.sync_copy(accs, acc_sh.at[pl.ds(sbase, SLICE)])

    def _eb(c):
        return pl.multiple_of(ebase + c * CHUNK, 8)

    def _prime():
        def _fill(i, _):
            colb_b[pl.ds(i * L, L)] = jnp.full((L,), NPAD - 1, jnp.int32)
            return 0
        lax.fori_loop(0, VPC, _fill, 0)
        pltpu.async_copy(valb_b, acc_sh.at[colb_b], sem_sc_b, add=True)

    def _gather(rb, vb):
        @plsc.parallel_loop(0, VPC, unroll=4)
        def _g(i):
            o = i * L
            vb[pl.ds(o, L)] = plsc.load_gather(q_local, [rb[pl.ds(o, L)]])

    _zero_acc_and_publish()
    plsc.subcore_barrier()

    _prime()
    pltpu.async_copy(col_hbm.at[pl.ds(_eb(0), CHUNK)], colb_a, sem_ld_a)

    def _ones_body(i, _):
        valb_a[pl.ds(i * L, L)] = zeros16 + 1.0
        valb_b[pl.ds(i * L, L)] = zeros16 + 1.0
        return 0
    lax.fori_loop(0, VPC, _ones_body, 0)

    def _deg_pair(p, _):
        c0 = 2 * p
        pltpu.make_async_copy(
            col_hbm.at[pl.ds(_eb(c0), CHUNK)], colb_a, sem_ld_a).wait()
        pltpu.make_async_copy(valb_b, acc_sh.at[colb_b], sem_sc_b).wait()
        pltpu.async_copy(col_hbm.at[pl.ds(_eb(c0 + 1), CHUNK)], colb_b, sem_ld_b)
        pltpu.async_copy(valb_a, acc_sh.at[colb_a], sem_sc_a, add=True)
        pltpu.make_async_copy(
            col_hbm.at[pl.ds(_eb(c0 + 1), CHUNK)], colb_b, sem_ld_b).wait()
        pltpu.make_async_copy(valb_a, acc_sh.at[colb_a], sem_sc_a).wait()
        pltpu.async_copy(col_hbm.at[pl.ds(_eb(c0 + 2), CHUNK)], colb_a, sem_ld_a)
        pltpu.async_copy(valb_b, acc_sh.at[colb_b], sem_sc_b, add=True)
        return 0
    lax.fori_loop(0, NPAIR, _deg_pair, 0)
    pltpu.make_async_copy(
        col_hbm.at[pl.ds(_eb(NCH), CHUNK)], colb_a, sem_ld_a).wait()
    pltpu.make_async_copy(valb_b, acc_sh.at[colb_b], sem_sc_b).wait()
    plsc.subcore_barrier()
    pltpu.sync_copy(acc_sh.at[pl.ds(sbase, SLICE)],
                    part_hbm.at[pl.ds(cid * NPAD + sbase, SLICE)])
    _gbarrier(1)

    pltpu.sync_copy(part_hbm.at[pl.ds(wbase, WSLICE)], accs.at[pl.ds(0, WSLICE)])
    pltpu.sync_copy(part_hbm.at[pl.ds(NPAD + wbase, WSLICE)], bufb)

    def _init_body(i, _):
        deg = accs[pl.ds(i * L, L)] + bufb[pl.ds(i * L, L)] + 1.0
        dinvs[pl.ds(i * L, L)] = _rsqrt_nr(deg)
        accs[pl.ds(i * L, L)] = zeros16 + inv_n
        return 0
    lax.fori_loop(0, NVW, _init_body, 0)
    pltpu.sync_copy(accs.at[pl.ds(0, WSLICE)], probs_hbm.at[pl.ds(wbase, WSLICE)])

    def _q0_body(i, _):
        accs[pl.ds(i * L, L)] = dinvs[pl.ds(i * L, L)] * inv_n
        return 0
    lax.fori_loop(0, NVW, _q0_body, 0)
    pltpu.sync_copy(accs.at[pl.ds(0, WSLICE)], q_hbm.at[pl.ds(wbase, WSLICE)])
    _zero_acc_and_publish()
    _gbarrier(2)
    pltpu.sync_copy(q_hbm, q_local)

    def _step_body(t, _):
        _prime()
        pltpu.async_copy(row_hbm.at[pl.ds(_eb(0), CHUNK)], rowb_a, sem_ld_a)
        pltpu.async_copy(col_hbm.at[pl.ds(_eb(0), CHUNK)], colb_a, sem_ld_a)

        def _pair(p, _p):
            c0 = 2 * p
            pltpu.make_async_copy(
                row_hbm.at[pl.ds(_eb(c0), CHUNK)], rowb_a, sem_ld_a).wait()
            pltpu.make_async_copy(
                col_hbm.at[pl.ds(_eb(c0), CHUNK)], colb_a, sem_ld_a).wait()
            pltpu.make_async_copy(valb_b, acc_sh.at[colb_b], sem_sc_b).wait()
            pltpu.async_copy(row_hbm.at[pl.ds(_eb(c0 + 1), CHUNK)], rowb_b, sem_ld_b)
            pltpu.async_copy(col_hbm.at[pl.ds(_eb(c0 + 1), CHUNK)], colb_b, sem_ld_b)
            _gather(rowb_a, valb_a)
            pltpu.async_copy(valb_a, acc_sh.at[colb_a], sem_sc_a, add=True)
            pltpu.make_async_copy(
                row_hbm.at[pl.ds(_eb(c0 + 1), CHUNK)], rowb_b, sem_ld_b).wait()
            pltpu.make_async_copy(
                col_hbm.at[pl.ds(_eb(c0 + 1), CHUNK)], colb_b, sem_ld_b).wait()
            _gather(rowb_b, valb_b)
            pltpu.make_async_copy(valb_a, acc_sh.at[colb_a], sem_sc_a).wait()
            pltpu.async_copy(row_hbm.at[pl.ds(_eb(c0 + 2), CHUNK)], rowb_a, sem_ld_a)
            pltpu.async_copy(col_hbm.at[pl.ds(_eb(c0 + 2), CHUNK)], colb_a, sem_ld_a)
            pltpu.async_copy(valb_b, acc_sh.at[colb_b], sem_sc_b, add=True)
            return 0
        lax.fori_loop(0, NPAIR, _pair, 0)
        pltpu.make_async_copy(
            row_hbm.at[pl.ds(_eb(NCH), CHUNK)], rowb_a, sem_ld_a).wait()
        pltpu.make_async_copy(
            col_hbm.at[pl.ds(_eb(NCH), CHUNK)], colb_a, sem_ld_a).wait()
        pltpu.make_async_copy(valb_b, acc_sh.at[colb_b], sem_sc_b).wait()
        plsc.subcore_barrier()
        pltpu.sync_copy(acc_sh.at[pl.ds(sbase, SLICE)],
                        part_hbm.at[pl.ds(cid * NPAD + sbase, SLICE)])
        _gbarrier(3 + 2 * t)

        pltpu.sync_copy(part_hbm.at[pl.ds(wbase, WSLICE)],
                        accs.at[pl.ds(0, WSLICE)])
        pltpu.sync_copy(part_hbm.at[pl.ds(NPAD + wbase, WSLICE)], bufb)

        def _upd_body(i, _u):
            s = accs[pl.ds(i * L, L)] + bufb[pl.ds(i * L, L)]
            qv = q_local[pl.ds(wbase + i * L, L)]
            dv = dinvs[pl.ds(i * L, L)]
            pv = qv / dv
            accs[pl.ds(i * L, L)] = 0.9 * (dv * (s + qv)) + 0.1 * pv
            return 0
        lax.fori_loop(0, NVW, _upd_body, 0)
        off = pl.multiple_of((t + 1) * NPAD + wbase, 8)
        pltpu.sync_copy(accs.at[pl.ds(0, WSLICE)], probs_hbm.at[pl.ds(off, WSLICE)])

        def _qn_body(i, _u):
            accs[pl.ds(i * L, L)] = accs[pl.ds(i * L, L)] * dinvs[pl.ds(i * L, L)]
            return 0
        lax.fori_loop(0, NVW, _qn_body, 0)
        pltpu.sync_copy(accs.at[pl.ds(0, WSLICE)], q_hbm.at[pl.ds(wbase, WSLICE)])
        _zero_acc_and_publish()
        _gbarrier(4 + 2 * t)
        pltpu.sync_copy(q_hbm, q_local)
        return 0
    lax.fori_loop(0, STEPS, _step_body, 0)


_walk = functools.partial(
    pl.kernel,
    out_type=[
        jax.ShapeDtypeStruct((WALK_LENGTH * NPAD,), jnp.float32),
        jax.ShapeDtypeStruct((NPAD,), jnp.float32),
        jax.ShapeDtypeStruct((NSC * NPAD,), jnp.float32),
    ],
    mesh=plsc.VectorSubcoreMesh(core_axis_name="c", subcore_axis_name="s"),
    compiler_params=pltpu.CompilerParams(needs_layout_passes=False),
    scratch_types=[
        pltpu.VMEM((NPAD,), jnp.float32),
        pltpu.VMEM((CHUNK,), jnp.int32),
        pltpu.VMEM((CHUNK,), jnp.int32),
        pltpu.VMEM((CHUNK,), jnp.float32),
        pltpu.VMEM((CHUNK,), jnp.int32),
        pltpu.VMEM((CHUNK,), jnp.int32),
        pltpu.VMEM((CHUNK,), jnp.float32),
        pltpu.VMEM((SLICE,), jnp.float32),
        pltpu.VMEM((WSLICE,), jnp.float32),
        pltpu.VMEM((WSLICE,), jnp.float32),
        pltpu.VMEM((L,), jnp.int32),
        pltpu.VMEM((FROWS,), jnp.int32),
        pltpu.VMEM_SHARED((NPAD,), jnp.float32),
        pltpu.SemaphoreType.DMA,
        pltpu.SemaphoreType.DMA,
        pltpu.SemaphoreType.DMA,
        pltpu.SemaphoreType.DMA,
    ],
)(_walk_body)


BN = 2048


def _linear_body(p_ref, w_ref, b_ref, o_ref):
    o_ref[...] = lax.dot_general(
        p_ref[...], w_ref[...], (((0,), (1,)), ((), ())),
        preferred_element_type=jnp.float32) + b_ref[...]


def _linear(probs2d, W, b2d):
    return pl.pallas_call(
        _linear_body,
        grid=(NPAD // BN,),
        in_specs=[
            pl.BlockSpec((WALK_LENGTH, BN), lambda i: (0, i)),
            pl.BlockSpec((EMBED_DIM, WALK_LENGTH), lambda i: (0, 0)),
            pl.BlockSpec((1, EMBED_DIM), lambda i: (0, 0)),
        ],
        out_specs=pl.BlockSpec((BN, EMBED_DIM), lambda i: (i, 0)),
        out_shape=jax.ShapeDtypeStruct((NPAD, EMBED_DIM), jnp.float32),
    )(probs2d, W, b2d)


def _zeroflags_body(o_ref):
    o_ref[...] = jnp.zeros((1, FROWS), jnp.int32)


def _zeroflags():
    out = pl.pallas_call(
        _zeroflags_body,
        out_shape=jax.ShapeDtypeStruct((1, FROWS), jnp.int32),
    )()
    return out.reshape(FROWS)


def kernel(edge_index, num_nodes, W, b):
    ei = edge_index.astype(jnp.int32)
    row, col = ei[0], ei[1]
    pad = E_ALLOC - row.shape[0]
    row_p = jnp.concatenate([row, jnp.zeros((pad,), jnp.int32)])
    col_p = jnp.concatenate([col, jnp.full((pad,), N_NODES, jnp.int32)])
    flags0 = _zeroflags()
    probs_flat, _, _ = _walk(row_p, col_p, flags0)
    probs2d = probs_flat.reshape(WALK_LENGTH, NPAD)
    out = _linear(probs2d, W.astype(jnp.float32),
                  b.astype(jnp.float32).reshape(1, EMBED_DIM))
    return out[:N_NODES]


# speedup vs baseline: 1.3662x; 1.0349x over previous
import functools

import jax
import jax.numpy as jnp
from jax import lax
from jax.experimental import pallas as pl
from jax.experimental.pallas import tpu as pltpu
from jax.experimental.pallas import tpu_sc as plsc

N_NODES = 100000
WALK_LENGTH = 16
EMBED_DIM = 16

NSC = 2
NTILES = 16
NW = NSC * NTILES
L = 16
NPAD = 102400
SLICE = NPAD // NTILES
WSLICE = NPAD // NW
CHUNK = 1024
VPC = CHUNK // L
NCH = 98
NPAIR = NCH // 2
EPT = CHUNK * NCH
E_PAD = EPT * NW
E_ALLOC = E_PAD + 2 * CHUNK
STEPS = WALK_LENGTH - 1
NVS = SLICE // L
NVW = WSLICE // L
FROWS = NW * L


def _rsqrt_nr(d):
    i = lax.bitcast_convert_type(d, jnp.int32)
    i = jnp.int32(0x5F3759DF) - (i >> 1)
    y = lax.bitcast_convert_type(i, jnp.float32)
    for _ in range(3):
        y = y * (1.5 - 0.5 * d * y * y)
    return y


def _walk_body(row_hbm, col_hbm, flags_hbm, probs_hbm, q_hbm, part_hbm,
               q_local, rowb_a, colb_a, valb_a, rowb_b, colb_b, valb_b,
               accs, bufb, dinvs, flagv, flagall, acc_sh,
               sem_ld_a, sem_ld_b, sem_sc_a, sem_sc_b):
    cid = lax.axis_index("c")
    tid = lax.axis_index("s")
    wid = cid * NTILES + tid
    sbase = tid * SLICE
    wbase = wid * WSLICE
    ebase = wid * EPT
    zeros16 = jnp.zeros((L,), jnp.float32)
    inv_n = jnp.float32(1.0 / N_NODES)

    def _gbarrier(k):
        flagv[...] = jnp.zeros((L,), jnp.int32) + k
        pltpu.sync_copy(flagv, flags_hbm.at[pl.ds(wid * L, L)])

        def _poll(_):
            pltpu.sync_copy(flags_hbm, flagall)
            m = flagall[pl.ds(0, L)]
            for r in range(1, NW):
                m = jnp.minimum(m, flagall[pl.ds(r * L, L)])
            return jnp.min(m)
        lax.while_loop(lambda m: m < k, _poll, jnp.int32(0))

    def _zero_acc_and_publish():
        @plsc.parallel_loop(0, NVS, unroll=4)
        def _zero_slice(i):
            accs[pl.ds(i * L, L)] = zeros16
        pltpu.sync_copy(accs, acc_sh.at[pl.ds(sbase, SLICE)])

    def _eb(c):
        return pl.multiple_of(ebase + c * CHUNK, 8)

    def _prime():
        def _fill(i, _):
            colb_b[pl.ds(i * L, L)] = jnp.full((L,), NPAD - 1, jnp.int32)
            return 0
        lax.fori_loop(0, VPC, _fill, 0)
        pltpu.async_copy(valb_b, acc_sh.at[colb_b], sem_sc_b, add=True)

    def _gather(rb, vb):
        @plsc.parallel_loop(0, VPC, unroll=4)
        def _g(i):
            o = i * L
            vb[pl.ds(o, L)] = plsc.load_gather(q_local, [rb[pl.ds(o, L)]])

    _zero_acc_and_publish()
    plsc.subcore_barrier()

    _prime()
    pltpu.async_copy(col_hbm.at[pl.ds(_eb(0), CHUNK)], colb_a, sem_ld_a)

    def _ones_body(i, _):
        valb_a[pl.ds(i * L, L)] = zeros16 + 1.0
        valb_b[pl.ds(i * L, L)] = zeros16 + 1.0
        return 0
    lax.fori_loop(0, VPC, _ones_body, 0)

    def _deg_pair(p, _):
        c0 = 2 * p
        pltpu.make_async_copy(
            col_hbm.at[pl.ds(_eb(c0), CHUNK)], colb_a, sem_ld_a).wait()
        pltpu.make_async_copy(valb_b, acc_sh.at[colb_b], sem_sc_b).wait()
        pltpu.async_copy(col_hbm.at[pl.ds(_eb(c0 + 1), CHUNK)], colb_b, sem_ld_b)
        pltpu.async_copy(valb_a, acc_sh.at[colb_a], sem_sc_a, add=True)
        pltpu.make_async_copy(
            col_hbm.at[pl.ds(_eb(c0 + 1), CHUNK)], colb_b, sem_ld_b).wait()
        pltpu.make_async_copy(valb_a, acc_sh.at[colb_a], sem_sc_a).wait()
        pltpu.async_copy(col_hbm.at[pl.ds(_eb(c0 + 2), CHUNK)], colb_a, sem_ld_a)
        pltpu.async_copy(valb_b, acc_sh.at[colb_b], sem_sc_b, add=True)
        return 0
    lax.fori_loop(0, NPAIR, _deg_pair, 0)
    pltpu.make_async_copy(
        col_hbm.at[pl.ds(_eb(NCH), CHUNK)], colb_a, sem_ld_a).wait()
    pltpu.make_async_copy(valb_b, acc_sh.at[colb_b], sem_sc_b).wait()
    plsc.subcore_barrier()
    pltpu.sync_copy(acc_sh.at[pl.ds(sbase, SLICE)],
                    part_hbm.at[pl.ds(cid * NPAD + sbase, SLICE)])
    _gbarrier(1)

    pltpu.sync_copy(part_hbm.at[pl.ds(wbase, WSLICE)], accs.at[pl.ds(0, WSLICE)])
    pltpu.sync_copy(part_hbm.at[pl.ds(NPAD + wbase, WSLICE)], bufb)

    def _init_body(i, _):
        deg = accs[pl.ds(i * L, L)] + bufb[pl.ds(i * L, L)] + 1.0
        dinvs[pl.ds(i * L, L)] = _rsqrt_nr(deg)
        accs[pl.ds(i * L, L)] = zeros16 + inv_n
        return 0
    lax.fori_loop(0, NVW, _init_body, 0)
    pltpu.sync_copy(accs.at[pl.ds(0, WSLICE)], probs_hbm.at[pl.ds(wbase, WSLICE)])

    def _q0_body(i, _):
        accs[pl.ds(i * L, L)] = dinvs[pl.ds(i * L, L)] * inv_n
        return 0
    lax.fori_loop(0, NVW, _q0_body, 0)
    pltpu.sync_copy(accs.at[pl.ds(0, WSLICE)], q_hbm.at[pl.ds(wbase, WSLICE)])
    _zero_acc_and_publish()
    _gbarrier(2)
    pltpu.sync_copy(q_hbm, q_local)

    def _step_body(t, _):
        _prime()
        pltpu.async_copy(row_hbm.at[pl.ds(_eb(0), CHUNK)], rowb_a, sem_ld_a)
        pltpu.async_copy(col_hbm.at[pl.ds(_eb(0), CHUNK)], colb_a, sem_ld_a)

        def _pair(p, _p):
            c0 = 2 * p
            pltpu.make_async_copy(
                row_hbm.at[pl.ds(_eb(c0), CHUNK)], rowb_a, sem_ld_a).wait()
            pltpu.make_async_copy(
                col_hbm.at[pl.ds(_eb(c0), CHUNK)], colb_a, sem_ld_a).wait()
            pltpu.make_async_copy(valb_b, acc_sh.at[colb_b], sem_sc_b).wait()
            pltpu.async_copy(row_hbm.at[pl.ds(_eb(c0 + 1), CHUNK)], rowb_b, sem_ld_b)
            pltpu.async_copy(col_hbm.at[pl.ds(_eb(c0 + 1), CHUNK)], colb_b, sem_ld_b)
            _gather(rowb_a, valb_a)
            pltpu.async_copy(valb_a, acc_sh.at[colb_a], sem_sc_a, add=True)
            pltpu.make_async_copy(
                row_hbm.at[pl.ds(_eb(c0 + 1), CHUNK)], rowb_b, sem_ld_b).wait()
            pltpu.make_async_copy(
                col_hbm.at[pl.ds(_eb(c0 + 1), CHUNK)], colb_b, sem_ld_b).wait()
            _gather(rowb_b, valb_b)
            pltpu.make_async_copy(valb_a, acc_sh.at[colb_a], sem_sc_a).wait()
            pltpu.async_copy(row_hbm.at[pl.ds(_eb(c0 + 2), CHUNK)], rowb_a, sem_ld_a)
            pltpu.async_copy(col_hbm.at[pl.ds(_eb(c0 + 2), CHUNK)], colb_a, sem_ld_a)
            pltpu.async_copy(valb_b, acc_sh.at[colb_b], sem_sc_b, add=True)
            return 0
        lax.fori_loop(0, NPAIR, _pair, 0)
        pltpu.make_async_copy(
            row_hbm.at[pl.ds(_eb(NCH), CHUNK)], rowb_a, sem_ld_a).wait()
        pltpu.make_async_copy(
            col_hbm.at[pl.ds(_eb(NCH), CHUNK)], colb_a, sem_ld_a).wait()
        pltpu.make_async_copy(valb_b, acc_sh.at[colb_b], sem_sc_b).wait()
        plsc.subcore_barrier()
        pltpu.sync_copy(acc_sh.at[pl.ds(sbase, SLICE)],
                        part_hbm.at[pl.ds(cid * NPAD + sbase, SLICE)])
        _gbarrier(3 + 2 * t)

        pltpu.sync_copy(part_hbm.at[pl.ds(wbase, WSLICE)],
                        accs.at[pl.ds(0, WSLICE)])
        pltpu.sync_copy(part_hbm.at[pl.ds(NPAD + wbase, WSLICE)], bufb)

        @plsc.parallel_loop(0, NVW, unroll=4)
        def _upd_body(i):
            s = accs[pl.ds(i * L, L)] + bufb[pl.ds(i * L, L)]
            qv = q_local[pl.ds(wbase + i * L, L)]
            dv = dinvs[pl.ds(i * L, L)]
            pv = qv / dv
            accs[pl.ds(i * L, L)] = 0.9 * (dv * (s + qv)) + 0.1 * pv
        off = pl.multiple_of((t + 1) * NPAD + wbase, 8)
        pltpu.sync_copy(accs.at[pl.ds(0, WSLICE)], probs_hbm.at[pl.ds(off, WSLICE)])

        @plsc.parallel_loop(0, NVW, unroll=4)
        def _qn_body(i):
            accs[pl.ds(i * L, L)] = accs[pl.ds(i * L, L)] * dinvs[pl.ds(i * L, L)]
        pltpu.sync_copy(accs.at[pl.ds(0, WSLICE)], q_hbm.at[pl.ds(wbase, WSLICE)])
        _zero_acc_and_publish()
        _gbarrier(4 + 2 * t)
        pltpu.sync_copy(q_hbm, q_local)
        return 0
    lax.fori_loop(0, STEPS, _step_body, 0)


_walk = functools.partial(
    pl.kernel,
    out_type=[
        jax.ShapeDtypeStruct((WALK_LENGTH * NPAD,), jnp.float32),
        jax.ShapeDtypeStruct((NPAD,), jnp.float32),
        jax.ShapeDtypeStruct((NSC * NPAD,), jnp.float32),
    ],
    mesh=plsc.VectorSubcoreMesh(core_axis_name="c", subcore_axis_name="s"),
    compiler_params=pltpu.CompilerParams(needs_layout_passes=False),
    scratch_types=[
        pltpu.VMEM((NPAD,), jnp.float32),
        pltpu.VMEM((CHUNK,), jnp.int32),
        pltpu.VMEM((CHUNK,), jnp.int32),
        pltpu.VMEM((CHUNK,), jnp.float32),
        pltpu.VMEM((CHUNK,), jnp.int32),
        pltpu.VMEM((CHUNK,), jnp.int32),
        pltpu.VMEM((CHUNK,), jnp.float32),
        pltpu.VMEM((SLICE,), jnp.float32),
        pltpu.VMEM((WSLICE,), jnp.float32),
        pltpu.VMEM((WSLICE,), jnp.float32),
        pltpu.VMEM((L,), jnp.int32),
        pltpu.VMEM((FROWS,), jnp.int32),
        pltpu.VMEM_SHARED((NPAD,), jnp.float32),
        pltpu.SemaphoreType.DMA,
        pltpu.SemaphoreType.DMA,
        pltpu.SemaphoreType.DMA,
        pltpu.SemaphoreType.DMA,
    ],
)(_walk_body)


BN = 2048


def _linear_body(p_ref, w_ref, b_ref, o_ref):
    o_ref[...] = lax.dot_general(
        p_ref[...], w_ref[...], (((0,), (1,)), ((), ())),
        preferred_element_type=jnp.float32) + b_ref[...]


def _linear(probs2d, W, b2d):
    return pl.pallas_call(
        _linear_body,
        grid=(NPAD // BN,),
        in_specs=[
            pl.BlockSpec((WALK_LENGTH, BN), lambda i: (0, i)),
            pl.BlockSpec((EMBED_DIM, WALK_LENGTH), lambda i: (0, 0)),
            pl.BlockSpec((1, EMBED_DIM), lambda i: (0, 0)),
        ],
        out_specs=pl.BlockSpec((BN, EMBED_DIM), lambda i: (i, 0)),
        out_shape=jax.ShapeDtypeStruct((NPAD, EMBED_DIM), jnp.float32),
    )(probs2d, W, b2d)


def _zeroflags_body(o_ref):
    o_ref[...] = jnp.zeros((1, FROWS), jnp.int32)


def _zeroflags():
    out = pl.pallas_call(
        _zeroflags_body,
        out_shape=jax.ShapeDtypeStruct((1, FROWS), jnp.int32),
    )()
    return out.reshape(FROWS)


def kernel(edge_index, num_nodes, W, b):
    ei = edge_index.astype(jnp.int32)
    row, col = ei[0], ei[1]
    pad = E_ALLOC - row.shape[0]
    row_p = jnp.concatenate([row, jnp.zeros((pad,), jnp.int32)])
    col_p = jnp.concatenate([col, jnp.full((pad,), N_NODES, jnp.int32)])
    flags0 = _zeroflags()
    probs_flat, _, _ = _walk(row_p, col_p, flags0)
    probs2d = probs_flat.reshape(WALK_LENGTH, NPAD)
    out = _linear(probs2d, W.astype(jnp.float32),
                  b.astype(jnp.float32).reshape(1, EMBED_DIM))
    return out[:N_NODES]
